# baseline (device time: 6817 ns/iter reference)
import jax
import jax.numpy as jnp
from jax import lax
from jax.experimental import pallas as pl
from jax.experimental.pallas import tpu as pltpu


def kernel(x):
    _, m, n = x.shape
    half = n // 2
    n_chunks = 2
    rows = m // n_chunks

    def body(
        x_ref,
        out_ref,
        xp_ref,
        xo_ref,
        send_ref,
        recv_ref,
        o_ref,
        in_sems,
        out_sems,
        send_sems,
        recv_sems,
    ):
        my_x = lax.axis_index("x")
        my_y = lax.axis_index("y")
        my_z = lax.axis_index("z")
        peer_x = 1 - my_x
        peer = (peer_x, my_y, my_z)

        barrier_sem = pltpu.get_barrier_semaphore()
        pl.semaphore_signal(
            barrier_sem, inc=1, device_id=peer,
            device_id_type=pl.DeviceIdType.MESH,
        )

        peer_cps = []
        for c in range(n_chunks):
            cp = pltpu.make_async_copy(
                x_ref.at[0, pl.ds(c * rows, rows), pl.ds(peer_x * half, half)],
                xp_ref.at[pl.ds(c * rows, rows), :],
                in_sems.at[c],
            )
            cp.start()
            peer_cps.append(cp)
        own_cp = pltpu.make_async_copy(
            x_ref.at[0, :, pl.ds(my_x * half, half)],
            xo_ref,
            in_sems.at[n_chunks],
        )
        own_cp.start()

        rdmas = []
        for c in range(n_chunks):
            peer_cps[c].wait()
            send_ref[pl.ds(c * rows, rows), :] = xp_ref[
                pl.ds(c * rows, rows), :
            ].astype(jnp.bfloat16)
            if c == 0:
                pl.semaphore_wait(barrier_sem, 1)
            rdma = pltpu.make_async_remote_copy(
                src_ref=send_ref.at[pl.ds(c * rows, rows), :],
                dst_ref=recv_ref.at[pl.ds(c * rows, rows), :],
                send_sem=send_sems.at[c],
                recv_sem=recv_sems.at[c],
                device_id=peer,
                device_id_type=pl.DeviceIdType.MESH,
            )
            rdma.start()
            rdmas.append(rdma)

        own_cp.wait()
        out_cps = []
        for c in range(n_chunks):
            rdmas[c].wait_recv()
            o_ref[pl.ds(c * rows, rows), :] = (
                xo_ref[pl.ds(c * rows, rows), :]
                + recv_ref[pl.ds(c * rows, rows), :].astype(jnp.float32)
            ).astype(jnp.bfloat16)
            cp = pltpu.make_async_copy(
                o_ref.at[pl.ds(c * rows, rows), :],
                out_ref.at[pl.ds(c * rows, rows), :],
                out_sems.at[c],
            )
            cp.start()
            out_cps.append(cp)

        for c in range(n_chunks):
            out_cps[c].wait()
            rdmas[c].wait_send()

    return pl.pallas_call(
        body,
        out_shape=jax.ShapeDtypeStruct((m, half), jnp.bfloat16),
        in_specs=[pl.BlockSpec(memory_space=pltpu.MemorySpace.HBM)],
        out_specs=pl.BlockSpec(memory_space=pltpu.MemorySpace.HBM),
        scratch_shapes=[
            pltpu.VMEM((m, half), jnp.float32),
            pltpu.VMEM((m, half), jnp.float32),
            pltpu.VMEM((m, half), jnp.bfloat16),
            pltpu.VMEM((m, half), jnp.bfloat16),
            pltpu.VMEM((m, half), jnp.bfloat16),
            pltpu.SemaphoreType.DMA((n_chunks + 1,)),
            pltpu.SemaphoreType.DMA((n_chunks,)),
            pltpu.SemaphoreType.DMA((n_chunks,)),
            pltpu.SemaphoreType.DMA((n_chunks,)),
        ],
        compiler_params=pltpu.CompilerParams(collective_id=0),
    )(x)


# device time: 6647 ns/iter; 1.0256x vs baseline; 1.0256x over previous
import jax
import jax.numpy as jnp
from jax import lax
from jax.experimental import pallas as pl
from jax.experimental.pallas import tpu as pltpu


def kernel(x):
    _, m, n = x.shape
    half = n // 2
    n_chunks = 4
    rows = m // n_chunks

    def body(x_ref, out_ref, send_ref, recv_ref, send_sems, recv_sems):
        my_x = lax.axis_index("x")
        my_y = lax.axis_index("y")
        my_z = lax.axis_index("z")
        peer_x = 1 - my_x
        peer = (peer_x, my_y, my_z)

        barrier_sem = pltpu.get_barrier_semaphore()
        pl.semaphore_signal(
            barrier_sem, inc=1, device_id=peer,
            device_id_type=pl.DeviceIdType.MESH,
        )

        rdmas = []
        for c in range(n_chunks):
            send_ref[pl.ds(c * rows, rows), :] = x_ref[
                0, pl.ds(c * rows, rows), pl.ds(peer_x * half, half)
            ].astype(jnp.bfloat16)
            if c == 0:
                pl.semaphore_wait(barrier_sem, 1)
            rdma = pltpu.make_async_remote_copy(
                src_ref=send_ref.at[pl.ds(c * rows, rows), :],
                dst_ref=recv_ref.at[pl.ds(c * rows, rows), :],
                send_sem=send_sems.at[c],
                recv_sem=recv_sems.at[c],
                device_id=peer,
                device_id_type=pl.DeviceIdType.MESH,
            )
            rdma.start()
            rdmas.append(rdma)

        for c in range(n_chunks):
            rdmas[c].wait_recv()
            out_ref[pl.ds(c * rows, rows), :] = (
                x_ref[0, pl.ds(c * rows, rows), pl.ds(my_x * half, half)]
                + recv_ref[pl.ds(c * rows, rows), :].astype(jnp.float32)
            ).astype(jnp.bfloat16)
        for c in range(n_chunks):
            rdmas[c].wait_send()

    return pl.pallas_call(
        body,
        out_shape=jax.ShapeDtypeStruct((m, half), jnp.bfloat16),
        in_specs=[pl.BlockSpec(memory_space=pltpu.VMEM)],
        out_specs=pl.BlockSpec(memory_space=pltpu.VMEM),
        scratch_shapes=[
            pltpu.VMEM((m, half), jnp.bfloat16),
            pltpu.VMEM((m, half), jnp.bfloat16),
            pltpu.SemaphoreType.DMA((n_chunks,)),
            pltpu.SemaphoreType.DMA((n_chunks,)),
        ],
        compiler_params=pltpu.CompilerParams(collective_id=0),
    )(x)
